# baseline (speedup 1.0000x reference)
import functools

import jax
import jax.numpy as jnp
from jax import lax
from jax.experimental import pallas as pl
from jax.experimental.pallas import tpu as pltpu
from jax.experimental.pallas import tpu_sc as plsc

N = 10000
E = 320000
D_IN = 128
H_MID = 64
C_OUT = 40

LANE = 128
EPAD = 327680
EROWS = EPAD // LANE
NPAD = 10240

NC = 2
NS = 16
TROWS_AGG = EROWS // (NC * NS)
TROWS_DEG = EROWS // NS
NROWS_TILE = NPAD // NS
ZROWS = 32

_MESH = plsc.VectorSubcoreMesh(core_axis_name="c", subcore_axis_name="s")
_SC_PARAMS = pltpu.CompilerParams(use_tc_tiling_on_sc=False)


@functools.partial(
    pl.kernel,
    out_type=jax.ShapeDtypeStruct((NPAD,), jnp.float32),
    mesh=_MESH,
    compiler_params=_SC_PARAMS,
    scratch_types=[
        pltpu.VMEM((TROWS_DEG, LANE), jnp.int32),
        pltpu.VMEM((TROWS_DEG, LANE), jnp.float32),
        pltpu.VMEM((NROWS_TILE,), jnp.float32),
        pltpu.VMEM_SHARED((NPAD,), jnp.float32),
        pltpu.SemaphoreType.DMA,
    ],
)
def _deg_kernel(dst_hbm, ew_hbm, deg_out, idx_v, ew_v, zbuf, acc, sem):
    c = lax.axis_index("c")
    s = lax.axis_index("s")

    zeros16 = jnp.zeros((16,), jnp.float32)

    def _zero(i, carry):
        zbuf[pl.ds(i * 16, 16)] = zeros16
        return carry

    lax.fori_loop(0, NROWS_TILE // 16, _zero, 0)
    pltpu.sync_copy(zbuf, acc.at[pl.ds(s * NROWS_TILE, NROWS_TILE)])
    plsc.subcore_barrier()

    @pl.when(c == 0)
    def _scatter():
        row0 = s * TROWS_DEG
        pltpu.sync_copy(dst_hbm.at[pl.ds(row0, TROWS_DEG)], idx_v)
        pltpu.sync_copy(ew_hbm.at[pl.ds(row0, TROWS_DEG)], ew_v)

        def _chunk(k, carry):
            cps = [
                pltpu.async_copy(
                    ew_v.at[k * 8 + j], acc.at[idx_v.at[k * 8 + j]], sem,
                    add=True)
                for j in range(8)
            ]
            for cp in cps:
                cp.wait()
            return carry

        lax.fori_loop(0, TROWS_DEG // 8, _chunk, 0)

    plsc.subcore_barrier()

    @pl.when(c == 0)
    def _writeout():
        sl = pl.ds(s * NROWS_TILE, NROWS_TILE)
        pltpu.sync_copy(acc.at[sl], deg_out.at[sl])


FPAD = 128
FW2 = (C_OUT + 15) // 16 * 16


def _make_agg_kernel_v2(F):
    FW = (F + 15) // 16 * 16
    nfv = FW // 16
    nk = TROWS_AGG

    @functools.partial(
        pl.kernel,
        out_type=jax.ShapeDtypeStruct((NC, NPAD, FPAD), jnp.float32),
        mesh=_MESH,
        compiler_params=_SC_PARAMS,
        scratch_types=[
            pltpu.VMEM((TROWS_AGG, LANE), jnp.int32),
            pltpu.VMEM((TROWS_AGG, LANE), jnp.int32),
            pltpu.VMEM((TROWS_AGG, LANE), jnp.float32),
            pltpu.VMEM((2, LANE, FW), jnp.float32),
            pltpu.VMEM((ZROWS, FW), jnp.float32),
            pltpu.VMEM_SHARED((NPAD, FW), jnp.float32),
            pltpu.SemaphoreType.DMA,
            pltpu.SemaphoreType.DMA,
            pltpu.SemaphoreType.DMA,
            pltpu.SemaphoreType.DMA,
        ],
    )
    def _agg(ht_hbm, src_hbm, dst_hbm, ew_hbm, out_hbm,
             src_v, dst_v, ew_v, rows, zbuf, acc, g0, g1, s0, s1):
        c = lax.axis_index("c")
        s = lax.axis_index("s")
        wid = c * NS + s
        gsem = (g0, g1)
        ssem = (s0, s1)
        zeros16 = jnp.zeros((16,), jnp.float32)

        def _fire_gather(k, b):
            pltpu.async_copy(
                ht_hbm.at[src_v.at[k]], rows.at[b], gsem[b])

        def _wait_gather(k, b):
            pltpu.make_async_copy(
                ht_hbm.at[src_v.at[k]], rows.at[b], gsem[b]).wait()

        def _fire_scatter(k, b):
            pltpu.async_copy(
                rows.at[b], acc.at[dst_v.at[k]], ssem[b], add=True)

        def _wait_scatter(k, b):
            pltpu.make_async_copy(
                rows.at[b], acc.at[dst_v.at[k]], ssem[b]).wait()

        def _scale(k, b):
            def body(g, carry):
                ew16 = ew_v[k, pl.ds(g * 16, 16)]
                for ii in range(16):
                    bc = lax.broadcast(ew16[ii], (16,))
                    r = g * 16 + ii
                    for f in range(nfv):
                        sl = pl.ds(f * 16, 16)
                        rows[b, r, sl] = rows[b, r, sl] * bc
                return carry

            lax.fori_loop(0, LANE // 16, body, 0)

        def _zero(i, carry):
            r = i // nfv
            f = i % nfv
            zbuf[r, pl.ds(f * 16, 16)] = zeros16
            return carry

        lax.fori_loop(0, ZROWS * nfv, _zero, 0)
        zcps = [
            pltpu.make_async_copy(
                zbuf, acc.at[pl.ds(s * NROWS_TILE + q * ZROWS, ZROWS)], g0)
            for q in range(NROWS_TILE // ZROWS)
        ]
        for cp in zcps:
            cp.start()
        row0 = wid * TROWS_AGG
        pltpu.sync_copy(src_hbm.at[pl.ds(row0, TROWS_AGG)], src_v)
        pltpu.sync_copy(dst_hbm.at[pl.ds(row0, TROWS_AGG)], dst_v)
        pltpu.sync_copy(ew_hbm.at[pl.ds(row0, TROWS_AGG)], ew_v)
        for cp in zcps:
            cp.wait()
        _fire_gather(0, 0)
        _fire_gather(1, 1)
        plsc.subcore_barrier()

        _wait_gather(0, 0)
        _scale(0, 0)
        _fire_scatter(0, 0)

        def _steady(k0, carry):
            for boff in range(2):
                k = 2 * k0 + 1 + boff
                b = (1 + boff) % 2
                bn = boff
                _wait_scatter(k - 1, bn)
                _fire_gather(k + 1, bn)
                _wait_gather(k, b)
                _scale(k, b)
                _fire_scatter(k, b)
            return carry

        lax.fori_loop(0, (nk - 2) // 2, _steady, 0)

        _wait_scatter(nk - 2, 0)
        _wait_gather(nk - 1, 1)
        _scale(nk - 1, 1)
        _fire_scatter(nk - 1, 1)
        _wait_scatter(nk - 1, 1)
        plsc.subcore_barrier()

        sl = pl.ds(s * NROWS_TILE, NROWS_TILE)
        pltpu.sync_copy(acc.at[sl], out_hbm.at[c, sl, pl.ds(0, FW)])

    return _agg


def _make_agg_kernel(F):
    cb_rows = 2
    cb = cb_rows * LANE

    @functools.partial(
        pl.kernel,
        out_type=jax.ShapeDtypeStruct((NC, NPAD, FPAD), jnp.float32),
        mesh=_MESH,
        compiler_params=_SC_PARAMS,
        scratch_types=[
            pltpu.VMEM((cb_rows, LANE), jnp.int32),
            pltpu.VMEM((cb_rows, LANE), jnp.int32),
            pltpu.VMEM((cb_rows, LANE), jnp.float32),
            pltpu.VMEM((cb, FPAD), jnp.float32),
            pltpu.VMEM((ZROWS, FPAD), jnp.float32),
            pltpu.VMEM_SHARED((NPAD, FPAD), jnp.float32),
            pltpu.SemaphoreType.DMA,
        ],
    )
    def _agg(ht_hbm, src_hbm, dst_hbm, ew_hbm, out_hbm,
             src_v, dst_v, ew_v, rows, zbuf, acc, sem):
        c = lax.axis_index("c")
        s = lax.axis_index("s")
        wid = c * NS + s

        zeros16 = jnp.zeros((16,), jnp.float32)

        def _zero(i, carry):
            r = i // (FPAD // 16)
            f = i % (FPAD // 16)
            zbuf[r, pl.ds(f * 16, 16)] = zeros16
            return carry

        lax.fori_loop(0, ZROWS * (FPAD // 16), _zero, 0)
        for q in range(NROWS_TILE // ZROWS):
            pltpu.sync_copy(
                zbuf, acc.at[pl.ds(s * NROWS_TILE + q * ZROWS, ZROWS)])
        plsc.subcore_barrier()

        row0 = wid * TROWS_AGG

        def _chunk(k, carry):
            base = row0 + k * cb_rows
            pltpu.sync_copy(src_hbm.at[pl.ds(base, cb_rows)], src_v)
            pltpu.sync_copy(dst_hbm.at[pl.ds(base, cb_rows)], dst_v)
            pltpu.sync_copy(ew_hbm.at[pl.ds(base, cb_rows)], ew_v)
            cps = [
                pltpu.async_copy(
                    ht_hbm.at[src_v.at[j]],
                    rows.at[pl.ds(j * LANE, LANE)], sem)
                for j in range(cb_rows)
            ]
            for cp in cps:
                cp.wait()

            def _scale(i, carry2):
                j = i % cb_rows
                g = i // cb_rows
                ew16 = ew_v[j, pl.ds(g * 16, 16)]
                for ii in range(16):
                    bc = lax.broadcast(ew16[ii], (16,))
                    r = j * LANE + g * 16 + ii
                    for f in range((F + 15) // 16):
                        sl = pl.ds(f * 16, 16)
                        rows[r, sl] = rows[r, sl] * bc
                return carry2

            lax.fori_loop(0, cb_rows * (LANE // 16), _scale, 0)

            cps = [
                pltpu.async_copy(
                    rows.at[pl.ds(j * LANE, LANE)],
                    acc.at[dst_v.at[j]], sem, add=True)
                for j in range(cb_rows)
            ]
            for cp in cps:
                cp.wait()
            return carry

        lax.fori_loop(0, TROWS_AGG // cb_rows, _chunk, 0)
        plsc.subcore_barrier()

        sl = pl.ds(s * NROWS_TILE, NROWS_TILE)
        pltpu.sync_copy(acc.at[sl], out_hbm.at[c, sl])

    return _agg


_agg64 = _make_agg_kernel_v2(H_MID)
_agg40 = _make_agg_kernel_v2(C_OUT)


def _tc1_body(deg_ref, x_ref, w1_ref, ht_ref, dinv_ref):
    deg = deg_ref[0:N, :] + 1.0
    dinv = jnp.where(deg > 0, lax.rsqrt(jnp.maximum(deg, 1e-12)), 0.0)
    h = jnp.dot(x_ref[...], w1_ref[...], preferred_element_type=jnp.float32)
    ht_ref[...] = dinv * h
    dinv_ref[...] = dinv


def _tc2_body(p_ref, ht1_ref, dinv_ref, b1_ref, w2_ref, ht2_ref):
    dinv = dinv_ref[...]
    agg = (p_ref[0, 0:N, 0:H_MID] + p_ref[1, 0:N, 0:H_MID]
           + ht1_ref[0:N, 0:H_MID])
    x2 = jnp.maximum(dinv * agg + b1_ref[...], 0.0)
    ht2 = dinv * jnp.dot(x2, w2_ref[...], preferred_element_type=jnp.float32)
    ht2_ref[...] = jnp.concatenate(
        [ht2, jnp.zeros((N, FW2 - C_OUT), jnp.float32)], axis=1)


def _tc3_body(q_ref, ht2_ref, dinv_ref, b2_ref, out_ref):
    z = (dinv_ref[...] * (q_ref[0, 0:N, 0:C_OUT] + q_ref[1, 0:N, 0:C_OUT]
                          + ht2_ref[0:N, 0:C_OUT])
         + b2_ref[...])
    m = jnp.max(z, axis=1, keepdims=True)
    zz = z - m
    out_ref[...] = zz - jnp.log(jnp.sum(jnp.exp(zz), axis=1, keepdims=True))


def _pad_edges(a, pad):
    return jnp.concatenate([a, pad]).reshape(EROWS, LANE)


_DEBUG_JNP_DEG = False
_DEBUG_JNP_AGG = False


def kernel(x, edge_index, edge_attr, W1, b1, W2, b2):
    pad_idx = jnp.arange(EPAD - E, dtype=jnp.int32) % N
    src2 = _pad_edges(edge_index[0], pad_idx)
    dst2 = _pad_edges(edge_index[1], pad_idx)
    ew2 = _pad_edges(edge_attr, jnp.zeros((EPAD - E,), jnp.float32))

    if _DEBUG_JNP_DEG:
        deg = jnp.zeros((NPAD,), jnp.float32).at[edge_index[1]].add(
            edge_attr).reshape(NPAD, 1)
    else:
        deg = _deg_kernel(dst2, ew2).reshape(NPAD, 1)

    def _jnp_agg(ht):
        f = ht.shape[1]
        agg = jnp.zeros((NPAD, f), jnp.float32).at[edge_index[1]].add(
            edge_attr[:, None] * ht[edge_index[0]])
        return jnp.stack([agg, jnp.zeros_like(agg)])

    ht1, dinv = pl.pallas_call(
        _tc1_body,
        out_shape=(jax.ShapeDtypeStruct((N, H_MID), jnp.float32),
                   jax.ShapeDtypeStruct((N, 1), jnp.float32)),
    )(deg, x, W1)

    p1 = _jnp_agg(ht1) if _DEBUG_JNP_AGG else _agg64(ht1, src2, dst2, ew2)

    ht2 = pl.pallas_call(
        _tc2_body,
        out_shape=jax.ShapeDtypeStruct((N, FW2), jnp.float32),
    )(p1, ht1, dinv, b1.reshape(1, H_MID), W2)

    p2 = _jnp_agg(ht2) if _DEBUG_JNP_AGG else _agg40(ht2, src2, dst2, ew2)

    return pl.pallas_call(
        _tc3_body,
        out_shape=jax.ShapeDtypeStruct((N, C_OUT), jnp.float32),
    )(p2, ht2, dinv, b2.reshape(1, C_OUT))


# speedup vs baseline: 1.2524x; 1.2524x over previous
import functools

import jax
import jax.numpy as jnp
from jax import lax
from jax.experimental import pallas as pl
from jax.experimental.pallas import tpu as pltpu
from jax.experimental.pallas import tpu_sc as plsc

N = 10000
E = 320000
D_IN = 128
H_MID = 64
C_OUT = 40

LANE = 128
EPAD = 327680
EROWS = EPAD // LANE
NPAD = 10240

NC = 2
NS = 16
TROWS_AGG = EROWS // (NC * NS)
TROWS_DEG = EROWS // NS
NROWS_TILE = NPAD // NS
ZROWS = 32

_MESH = plsc.VectorSubcoreMesh(core_axis_name="c", subcore_axis_name="s")
_SC_PARAMS = pltpu.CompilerParams(use_tc_tiling_on_sc=False)


@functools.partial(
    pl.kernel,
    out_type=jax.ShapeDtypeStruct((NPAD,), jnp.float32),
    mesh=_MESH,
    compiler_params=_SC_PARAMS,
    scratch_types=[
        pltpu.VMEM((TROWS_DEG, LANE), jnp.int32),
        pltpu.VMEM((TROWS_DEG, LANE), jnp.float32),
        pltpu.VMEM((NROWS_TILE,), jnp.float32),
        pltpu.VMEM_SHARED((NPAD,), jnp.float32),
        pltpu.SemaphoreType.DMA,
    ],
)
def _deg_kernel(dst_hbm, ew_hbm, deg_out, idx_v, ew_v, zbuf, acc, sem):
    c = lax.axis_index("c")
    s = lax.axis_index("s")

    zeros16 = jnp.zeros((16,), jnp.float32)

    def _zero(i, carry):
        zbuf[pl.ds(i * 16, 16)] = zeros16
        return carry

    lax.fori_loop(0, NROWS_TILE // 16, _zero, 0)
    pltpu.sync_copy(zbuf, acc.at[pl.ds(s * NROWS_TILE, NROWS_TILE)])
    plsc.subcore_barrier()

    @pl.when(c == 0)
    def _scatter():
        row0 = s * TROWS_DEG
        pltpu.sync_copy(dst_hbm.at[pl.ds(row0, TROWS_DEG)], idx_v)
        pltpu.sync_copy(ew_hbm.at[pl.ds(row0, TROWS_DEG)], ew_v)

        def _chunk(k, carry):
            cps = [
                pltpu.async_copy(
                    ew_v.at[k * 8 + j], acc.at[idx_v.at[k * 8 + j]], sem,
                    add=True)
                for j in range(8)
            ]
            for cp in cps:
                cp.wait()
            return carry

        lax.fori_loop(0, TROWS_DEG // 8, _chunk, 0)

    plsc.subcore_barrier()

    @pl.when(c == 0)
    def _writeout():
        sl = pl.ds(s * NROWS_TILE, NROWS_TILE)
        pltpu.sync_copy(acc.at[sl], deg_out.at[sl])


FPAD = 128
FW2 = (C_OUT + 15) // 16 * 16


def _make_agg_kernel_v2(F):
    FW = (F + 15) // 16 * 16
    nfv = FW // 16
    nk = TROWS_AGG

    @functools.partial(
        pl.kernel,
        out_type=jax.ShapeDtypeStruct((NC, NPAD, FPAD), jnp.float32),
        mesh=_MESH,
        compiler_params=_SC_PARAMS,
        scratch_types=[
            pltpu.VMEM((TROWS_AGG, LANE), jnp.int32),
            pltpu.VMEM((TROWS_AGG, LANE), jnp.int32),
            pltpu.VMEM((TROWS_AGG, LANE), jnp.float32),
            pltpu.VMEM((4, LANE, FW), jnp.float32),
            pltpu.VMEM((ZROWS, FW), jnp.float32),
            pltpu.VMEM_SHARED((NPAD, FW), jnp.float32),
            pltpu.SemaphoreType.DMA,
            pltpu.SemaphoreType.DMA,
            pltpu.SemaphoreType.DMA,
            pltpu.SemaphoreType.DMA,
            pltpu.SemaphoreType.DMA,
            pltpu.SemaphoreType.DMA,
            pltpu.SemaphoreType.DMA,
            pltpu.SemaphoreType.DMA,
        ],
    )
    def _agg(ht_hbm, src_hbm, dst_hbm, ew_hbm, out_hbm,
             src_v, dst_v, ew_v, rows, zbuf, acc,
             g0, g1, g2, g3, s0, s1, s2, s3):
        c = lax.axis_index("c")
        s = lax.axis_index("s")
        wid = c * NS + s
        gsem = (g0, g1, g2, g3)
        ssem = (s0, s1, s2, s3)
        zeros16 = jnp.zeros((16,), jnp.float32)

        def _fire_gather(k, b):
            pltpu.async_copy(
                ht_hbm.at[src_v.at[k]], rows.at[b], gsem[b])

        def _wait_gather(k, b):
            pltpu.make_async_copy(
                ht_hbm.at[src_v.at[k]], rows.at[b], gsem[b]).wait()

        def _fire_scatter(k, b):
            pltpu.async_copy(
                rows.at[b], acc.at[dst_v.at[k]], ssem[b], add=True)

        def _wait_scatter(k, b):
            pltpu.make_async_copy(
                rows.at[b], acc.at[dst_v.at[k]], ssem[b]).wait()

        def _scale(k, b):
            def body(g, carry):
                ew16 = ew_v[k, pl.ds(g * 16, 16)]
                for ii in range(16):
                    bc = lax.broadcast(ew16[ii], (16,))
                    r = g * 16 + ii
                    for f in range(nfv):
                        sl = pl.ds(f * 16, 16)
                        rows[b, r, sl] = rows[b, r, sl] * bc
                return carry

            lax.fori_loop(0, LANE // 16, body, 0, unroll=2)

        def _zero(i, carry):
            r = i // nfv
            f = i % nfv
            zbuf[r, pl.ds(f * 16, 16)] = zeros16
            return carry

        lax.fori_loop(0, ZROWS * nfv, _zero, 0)
        zcps = [
            pltpu.make_async_copy(
                zbuf, acc.at[pl.ds(s * NROWS_TILE + q * ZROWS, ZROWS)], g0)
            for q in range(NROWS_TILE // ZROWS)
        ]
        for cp in zcps:
            cp.start()
        row0 = wid * TROWS_AGG
        pltpu.sync_copy(src_hbm.at[pl.ds(row0, TROWS_AGG)], src_v)
        pltpu.sync_copy(dst_hbm.at[pl.ds(row0, TROWS_AGG)], dst_v)
        pltpu.sync_copy(ew_hbm.at[pl.ds(row0, TROWS_AGG)], ew_v)
        for cp in zcps:
            cp.wait()
        _fire_gather(0, 0)
        _fire_gather(1, 1)
        plsc.subcore_barrier()

        _fire_gather(2, 2)
        _wait_gather(0, 0)
        _scale(0, 0)
        _fire_scatter(0, 0)

        _fire_gather(3, 3)
        _wait_gather(1, 1)
        _scale(1, 1)
        _fire_scatter(1, 1)

        def _steady(k0, carry):
            for boff in range(4):
                k = 4 * k0 + 2 + boff
                b = (2 + boff) % 4
                bn = boff
                _wait_scatter(k - 2, bn)
                _fire_gather(k + 2, bn)
                _wait_gather(k, b)
                _scale(k, b)
                _fire_scatter(k, b)
            return carry

        lax.fori_loop(0, (nk - 4) // 4, _steady, 0)

        _wait_scatter(nk - 4, 0)
        _wait_gather(nk - 2, 2)
        _scale(nk - 2, 2)
        _fire_scatter(nk - 2, 2)

        _wait_scatter(nk - 3, 1)
        _wait_gather(nk - 1, 3)
        _scale(nk - 1, 3)
        _fire_scatter(nk - 1, 3)

        _wait_scatter(nk - 2, 2)
        _wait_scatter(nk - 1, 3)
        plsc.subcore_barrier()

        sl = pl.ds(s * NROWS_TILE, NROWS_TILE)
        pltpu.sync_copy(acc.at[sl], out_hbm.at[c, sl, pl.ds(0, FW)])

    return _agg


def _make_agg_kernel(F):
    cb_rows = 2
    cb = cb_rows * LANE

    @functools.partial(
        pl.kernel,
        out_type=jax.ShapeDtypeStruct((NC, NPAD, FPAD), jnp.float32),
        mesh=_MESH,
        compiler_params=_SC_PARAMS,
        scratch_types=[
            pltpu.VMEM((cb_rows, LANE), jnp.int32),
            pltpu.VMEM((cb_rows, LANE), jnp.int32),
            pltpu.VMEM((cb_rows, LANE), jnp.float32),
            pltpu.VMEM((cb, FPAD), jnp.float32),
            pltpu.VMEM((ZROWS, FPAD), jnp.float32),
            pltpu.VMEM_SHARED((NPAD, FPAD), jnp.float32),
            pltpu.SemaphoreType.DMA,
        ],
    )
    def _agg(ht_hbm, src_hbm, dst_hbm, ew_hbm, out_hbm,
             src_v, dst_v, ew_v, rows, zbuf, acc, sem):
        c = lax.axis_index("c")
        s = lax.axis_index("s")
        wid = c * NS + s

        zeros16 = jnp.zeros((16,), jnp.float32)

        def _zero(i, carry):
            r = i // (FPAD // 16)
            f = i % (FPAD // 16)
            zbuf[r, pl.ds(f * 16, 16)] = zeros16
            return carry

        lax.fori_loop(0, ZROWS * (FPAD // 16), _zero, 0)
        for q in range(NROWS_TILE // ZROWS):
            pltpu.sync_copy(
                zbuf, acc.at[pl.ds(s * NROWS_TILE + q * ZROWS, ZROWS)])
        plsc.subcore_barrier()

        row0 = wid * TROWS_AGG

        def _chunk(k, carry):
            base = row0 + k * cb_rows
            pltpu.sync_copy(src_hbm.at[pl.ds(base, cb_rows)], src_v)
            pltpu.sync_copy(dst_hbm.at[pl.ds(base, cb_rows)], dst_v)
            pltpu.sync_copy(ew_hbm.at[pl.ds(base, cb_rows)], ew_v)
            cps = [
                pltpu.async_copy(
                    ht_hbm.at[src_v.at[j]],
                    rows.at[pl.ds(j * LANE, LANE)], sem)
                for j in range(cb_rows)
            ]
            for cp in cps:
                cp.wait()

            def _scale(i, carry2):
                j = i % cb_rows
                g = i // cb_rows
                ew16 = ew_v[j, pl.ds(g * 16, 16)]
                for ii in range(16):
                    bc = lax.broadcast(ew16[ii], (16,))
                    r = j * LANE + g * 16 + ii
                    for f in range((F + 15) // 16):
                        sl = pl.ds(f * 16, 16)
                        rows[r, sl] = rows[r, sl] * bc
                return carry2

            lax.fori_loop(0, cb_rows * (LANE // 16), _scale, 0)

            cps = [
                pltpu.async_copy(
                    rows.at[pl.ds(j * LANE, LANE)],
                    acc.at[dst_v.at[j]], sem, add=True)
                for j in range(cb_rows)
            ]
            for cp in cps:
                cp.wait()
            return carry

        lax.fori_loop(0, TROWS_AGG // cb_rows, _chunk, 0)
        plsc.subcore_barrier()

        sl = pl.ds(s * NROWS_TILE, NROWS_TILE)
        pltpu.sync_copy(acc.at[sl], out_hbm.at[c, sl])

    return _agg


_agg64 = _make_agg_kernel_v2(H_MID)
_agg40 = _make_agg_kernel_v2(C_OUT)


def _tc1_body(deg_ref, x_ref, w1_ref, ht_ref, dinv_ref):
    deg = deg_ref[0:N, :] + 1.0
    dinv = jnp.where(deg > 0, lax.rsqrt(jnp.maximum(deg, 1e-12)), 0.0)
    h = jnp.dot(x_ref[...], w1_ref[...], preferred_element_type=jnp.float32)
    ht_ref[...] = dinv * h
    dinv_ref[...] = dinv


def _tc2_body(p_ref, ht1_ref, dinv_ref, b1_ref, w2_ref, ht2_ref):
    dinv = dinv_ref[...]
    agg = (p_ref[0, 0:N, 0:H_MID] + p_ref[1, 0:N, 0:H_MID]
           + ht1_ref[0:N, 0:H_MID])
    x2 = jnp.maximum(dinv * agg + b1_ref[...], 0.0)
    ht2 = dinv * jnp.dot(x2, w2_ref[...], preferred_element_type=jnp.float32)
    ht2_ref[...] = jnp.concatenate(
        [ht2, jnp.zeros((N, FW2 - C_OUT), jnp.float32)], axis=1)


def _tc3_body(q_ref, ht2_ref, dinv_ref, b2_ref, out_ref):
    z = (dinv_ref[...] * (q_ref[0, 0:N, 0:C_OUT] + q_ref[1, 0:N, 0:C_OUT]
                          + ht2_ref[0:N, 0:C_OUT])
         + b2_ref[...])
    m = jnp.max(z, axis=1, keepdims=True)
    zz = z - m
    out_ref[...] = zz - jnp.log(jnp.sum(jnp.exp(zz), axis=1, keepdims=True))


def _pad_edges(a, pad):
    return jnp.concatenate([a, pad]).reshape(EROWS, LANE)


_DEBUG_JNP_DEG = False
_DEBUG_JNP_AGG = False


def kernel(x, edge_index, edge_attr, W1, b1, W2, b2):
    pad_idx = jnp.arange(EPAD - E, dtype=jnp.int32) % N
    src2 = _pad_edges(edge_index[0], pad_idx)
    dst2 = _pad_edges(edge_index[1], pad_idx)
    ew2 = _pad_edges(edge_attr, jnp.zeros((EPAD - E,), jnp.float32))

    if _DEBUG_JNP_DEG:
        deg = jnp.zeros((NPAD,), jnp.float32).at[edge_index[1]].add(
            edge_attr).reshape(NPAD, 1)
    else:
        deg = _deg_kernel(dst2, ew2).reshape(NPAD, 1)

    def _jnp_agg(ht):
        f = ht.shape[1]
        agg = jnp.zeros((NPAD, f), jnp.float32).at[edge_index[1]].add(
            edge_attr[:, None] * ht[edge_index[0]])
        return jnp.stack([agg, jnp.zeros_like(agg)])

    ht1, dinv = pl.pallas_call(
        _tc1_body,
        out_shape=(jax.ShapeDtypeStruct((N, H_MID), jnp.float32),
                   jax.ShapeDtypeStruct((N, 1), jnp.float32)),
    )(deg, x, W1)

    p1 = _jnp_agg(ht1) if _DEBUG_JNP_AGG else _agg64(ht1, src2, dst2, ew2)

    ht2 = pl.pallas_call(
        _tc2_body,
        out_shape=jax.ShapeDtypeStruct((N, FW2), jnp.float32),
    )(p1, ht1, dinv, b1.reshape(1, H_MID), W2)

    p2 = _jnp_agg(ht2) if _DEBUG_JNP_AGG else _agg40(ht2, src2, dst2, ew2)

    return pl.pallas_call(
        _tc3_body,
        out_shape=jax.ShapeDtypeStruct((N, C_OUT), jnp.float32),
    )(p2, ht2, dinv, b2.reshape(1, C_OUT))


# speedup vs baseline: 1.2580x; 1.0045x over previous
import functools

import jax
import jax.numpy as jnp
from jax import lax
from jax.experimental import pallas as pl
from jax.experimental.pallas import tpu as pltpu
from jax.experimental.pallas import tpu_sc as plsc

N = 10000
E = 320000
D_IN = 128
H_MID = 64
C_OUT = 40

LANE = 128
EPAD = 327680
EROWS = EPAD // LANE
NPAD = 10240

NC = 2
NS = 16
TROWS_AGG = EROWS // (NC * NS)
TROWS_DEG = EROWS // NS
NROWS_TILE = NPAD // NS
ZROWS = 32

_MESH = plsc.VectorSubcoreMesh(core_axis_name="c", subcore_axis_name="s")
_SC_PARAMS = pltpu.CompilerParams(use_tc_tiling_on_sc=False)


@functools.partial(
    pl.kernel,
    out_type=(jax.ShapeDtypeStruct((NPAD,), jnp.float32),
              jax.ShapeDtypeStruct((NPAD,), jnp.float32)),
    mesh=_MESH,
    compiler_params=_SC_PARAMS,
    scratch_types=[
        pltpu.VMEM((TROWS_AGG, LANE), jnp.int32),
        pltpu.VMEM((TROWS_AGG, LANE), jnp.float32),
        pltpu.VMEM((NROWS_TILE,), jnp.float32),
        pltpu.VMEM_SHARED((NPAD,), jnp.float32),
        pltpu.SemaphoreType.DMA,
    ],
)
def _deg_kernel(dst_hbm, ew_hbm, deg0_out, deg1_out, idx_v, ew_v, zbuf, acc,
                sem):
    c = lax.axis_index("c")
    s = lax.axis_index("s")
    wid = c * NS + s

    zeros16 = jnp.zeros((16,), jnp.float32)

    def _zero(i, carry):
        zbuf[pl.ds(i * 16, 16)] = zeros16
        return carry

    lax.fori_loop(0, NROWS_TILE // 16, _zero, 0)
    pltpu.sync_copy(zbuf, acc.at[pl.ds(s * NROWS_TILE, NROWS_TILE)])
    row0 = wid * TROWS_AGG
    pltpu.sync_copy(dst_hbm.at[pl.ds(row0, TROWS_AGG)], idx_v)
    pltpu.sync_copy(ew_hbm.at[pl.ds(row0, TROWS_AGG)], ew_v)
    plsc.subcore_barrier()

    def _chunk(k, carry):
        cps = [
            pltpu.async_copy(
                ew_v.at[k * 8 + j], acc.at[idx_v.at[k * 8 + j]], sem,
                add=True)
            for j in range(8)
        ]
        for cp in cps:
            cp.wait()
        return carry

    lax.fori_loop(0, TROWS_AGG // 8, _chunk, 0)
    plsc.subcore_barrier()

    sl = pl.ds(s * NROWS_TILE, NROWS_TILE)

    @pl.when(c == 0)
    def _writeout0():
        pltpu.sync_copy(acc.at[sl], deg0_out.at[sl])

    @pl.when(c == 1)
    def _writeout1():
        pltpu.sync_copy(acc.at[sl], deg1_out.at[sl])


FPAD = 128
FW2 = (C_OUT + 15) // 16 * 16


def _make_agg_kernel_v2(F):
    FW = (F + 15) // 16 * 16
    nfv = FW // 16
    nk = TROWS_AGG

    @functools.partial(
        pl.kernel,
        out_type=jax.ShapeDtypeStruct((NC, NPAD, FPAD), jnp.float32),
        mesh=_MESH,
        compiler_params=_SC_PARAMS,
        scratch_types=[
            pltpu.VMEM((TROWS_AGG, LANE), jnp.int32),
            pltpu.VMEM((TROWS_AGG, LANE), jnp.int32),
            pltpu.VMEM((TROWS_AGG, LANE), jnp.float32),
            pltpu.VMEM((4, LANE, FW), jnp.float32),
            pltpu.VMEM((ZROWS, FW), jnp.float32),
            pltpu.VMEM_SHARED((NPAD, FW), jnp.float32),
            pltpu.SemaphoreType.DMA,
            pltpu.SemaphoreType.DMA,
            pltpu.SemaphoreType.DMA,
            pltpu.SemaphoreType.DMA,
            pltpu.SemaphoreType.DMA,
            pltpu.SemaphoreType.DMA,
            pltpu.SemaphoreType.DMA,
            pltpu.SemaphoreType.DMA,
        ],
    )
    def _agg(ht_hbm, src_hbm, dst_hbm, ew_hbm, out_hbm,
             src_v, dst_v, ew_v, rows, zbuf, acc,
             g0, g1, g2, g3, s0, s1, s2, s3):
        c = lax.axis_index("c")
        s = lax.axis_index("s")
        wid = c * NS + s
        gsem = (g0, g1, g2, g3)
        ssem = (s0, s1, s2, s3)
        zeros16 = jnp.zeros((16,), jnp.float32)

        def _fire_gather(k, b):
            pltpu.async_copy(
                ht_hbm.at[src_v.at[k]], rows.at[b], gsem[b])

        def _wait_gather(k, b):
            pltpu.make_async_copy(
                ht_hbm.at[src_v.at[k]], rows.at[b], gsem[b]).wait()

        def _fire_scatter(k, b):
            pltpu.async_copy(
                rows.at[b], acc.at[dst_v.at[k]], ssem[b], add=True)

        def _wait_scatter(k, b):
            pltpu.make_async_copy(
                rows.at[b], acc.at[dst_v.at[k]], ssem[b]).wait()

        def _scale(k, b):
            def body(g, carry):
                ew16 = ew_v[k, pl.ds(g * 16, 16)]
                for ii in range(16):
                    bc = lax.broadcast(ew16[ii], (16,))
                    r = g * 16 + ii
                    for f in range(nfv):
                        sl = pl.ds(f * 16, 16)
                        rows[b, r, sl] = rows[b, r, sl] * bc
                return carry

            lax.fori_loop(0, LANE // 16, body, 0, unroll=2)

        def _zero(i, carry):
            r = i // nfv
            f = i % nfv
            zbuf[r, pl.ds(f * 16, 16)] = zeros16
            return carry

        lax.fori_loop(0, ZROWS * nfv, _zero, 0)
        zcps = [
            pltpu.make_async_copy(
                zbuf, acc.at[pl.ds(s * NROWS_TILE + q * ZROWS, ZROWS)], g0)
            for q in range(NROWS_TILE // ZROWS)
        ]
        for cp in zcps:
            cp.start()
        row0 = wid * TROWS_AGG
        pltpu.sync_copy(src_hbm.at[pl.ds(row0, TROWS_AGG)], src_v)
        pltpu.sync_copy(dst_hbm.at[pl.ds(row0, TROWS_AGG)], dst_v)
        pltpu.sync_copy(ew_hbm.at[pl.ds(row0, TROWS_AGG)], ew_v)
        for cp in zcps:
            cp.wait()
        _fire_gather(0, 0)
        _fire_gather(1, 1)
        plsc.subcore_barrier()

        _fire_gather(2, 2)
        _wait_gather(0, 0)
        _scale(0, 0)
        _fire_scatter(0, 0)

        _fire_gather(3, 3)
        _wait_gather(1, 1)
        _scale(1, 1)
        _fire_scatter(1, 1)

        def _steady(k0, carry):
            for boff in range(4):
                k = 4 * k0 + 2 + boff
                b = (2 + boff) % 4
                bn = boff
                _wait_scatter(k - 2, bn)
                _fire_gather(k + 2, bn)
                _wait_gather(k, b)
                _scale(k, b)
                _fire_scatter(k, b)
            return carry

        lax.fori_loop(0, (nk - 4) // 4, _steady, 0)

        _wait_scatter(nk - 4, 0)
        _wait_gather(nk - 2, 2)
        _scale(nk - 2, 2)
        _fire_scatter(nk - 2, 2)

        _wait_scatter(nk - 3, 1)
        _wait_gather(nk - 1, 3)
        _scale(nk - 1, 3)
        _fire_scatter(nk - 1, 3)

        _wait_scatter(nk - 2, 2)
        _wait_scatter(nk - 1, 3)
        plsc.subcore_barrier()

        sl = pl.ds(s * NROWS_TILE, NROWS_TILE)
        pltpu.sync_copy(acc.at[sl], out_hbm.at[c, sl, pl.ds(0, FW)])

    return _agg


def _make_agg_kernel(F):
    cb_rows = 2
    cb = cb_rows * LANE

    @functools.partial(
        pl.kernel,
        out_type=jax.ShapeDtypeStruct((NC, NPAD, FPAD), jnp.float32),
        mesh=_MESH,
        compiler_params=_SC_PARAMS,
        scratch_types=[
            pltpu.VMEM((cb_rows, LANE), jnp.int32),
            pltpu.VMEM((cb_rows, LANE), jnp.int32),
            pltpu.VMEM((cb_rows, LANE), jnp.float32),
            pltpu.VMEM((cb, FPAD), jnp.float32),
            pltpu.VMEM((ZROWS, FPAD), jnp.float32),
            pltpu.VMEM_SHARED((NPAD, FPAD), jnp.float32),
            pltpu.SemaphoreType.DMA,
        ],
    )
    def _agg(ht_hbm, src_hbm, dst_hbm, ew_hbm, out_hbm,
             src_v, dst_v, ew_v, rows, zbuf, acc, sem):
        c = lax.axis_index("c")
        s = lax.axis_index("s")
        wid = c * NS + s

        zeros16 = jnp.zeros((16,), jnp.float32)

        def _zero(i, carry):
            r = i // (FPAD // 16)
            f = i % (FPAD // 16)
            zbuf[r, pl.ds(f * 16, 16)] = zeros16
            return carry

        lax.fori_loop(0, ZROWS * (FPAD // 16), _zero, 0)
        for q in range(NROWS_TILE // ZROWS):
            pltpu.sync_copy(
                zbuf, acc.at[pl.ds(s * NROWS_TILE + q * ZROWS, ZROWS)])
        plsc.subcore_barrier()

        row0 = wid * TROWS_AGG

        def _chunk(k, carry):
            base = row0 + k * cb_rows
            pltpu.sync_copy(src_hbm.at[pl.ds(base, cb_rows)], src_v)
            pltpu.sync_copy(dst_hbm.at[pl.ds(base, cb_rows)], dst_v)
            pltpu.sync_copy(ew_hbm.at[pl.ds(base, cb_rows)], ew_v)
            cps = [
                pltpu.async_copy(
                    ht_hbm.at[src_v.at[j]],
                    rows.at[pl.ds(j * LANE, LANE)], sem)
                for j in range(cb_rows)
            ]
            for cp in cps:
                cp.wait()

            def _scale(i, carry2):
                j = i % cb_rows
                g = i // cb_rows
                ew16 = ew_v[j, pl.ds(g * 16, 16)]
                for ii in range(16):
                    bc = lax.broadcast(ew16[ii], (16,))
                    r = j * LANE + g * 16 + ii
                    for f in range((F + 15) // 16):
                        sl = pl.ds(f * 16, 16)
                        rows[r, sl] = rows[r, sl] * bc
                return carry2

            lax.fori_loop(0, cb_rows * (LANE // 16), _scale, 0)

            cps = [
                pltpu.async_copy(
                    rows.at[pl.ds(j * LANE, LANE)],
                    acc.at[dst_v.at[j]], sem, add=True)
                for j in range(cb_rows)
            ]
            for cp in cps:
                cp.wait()
            return carry

        lax.fori_loop(0, TROWS_AGG // cb_rows, _chunk, 0)
        plsc.subcore_barrier()

        sl = pl.ds(s * NROWS_TILE, NROWS_TILE)
        pltpu.sync_copy(acc.at[sl], out_hbm.at[c, sl])

    return _agg


_agg64 = _make_agg_kernel_v2(H_MID)
_agg40 = _make_agg_kernel_v2(C_OUT)


def _tc1_body(deg0_ref, deg1_ref, x_ref, w1_ref, ht_ref, dinv_ref):
    deg = deg0_ref[0:N, :] + deg1_ref[0:N, :] + 1.0
    dinv = jnp.where(deg > 0, lax.rsqrt(jnp.maximum(deg, 1e-12)), 0.0)
    h = jnp.dot(x_ref[...], w1_ref[...], preferred_element_type=jnp.float32)
    ht_ref[...] = dinv * h
    dinv_ref[...] = dinv


def _tc2_body(p_ref, ht1_ref, dinv_ref, b1_ref, w2_ref, ht2_ref):
    dinv = dinv_ref[...]
    agg = (p_ref[0, 0:N, 0:H_MID] + p_ref[1, 0:N, 0:H_MID]
           + ht1_ref[0:N, 0:H_MID])
    x2 = jnp.maximum(dinv * agg + b1_ref[...], 0.0)
    ht2 = dinv * jnp.dot(x2, w2_ref[...], preferred_element_type=jnp.float32)
    ht2_ref[...] = jnp.concatenate(
        [ht2, jnp.zeros((N, FW2 - C_OUT), jnp.float32)], axis=1)


def _tc3_body(q_ref, ht2_ref, dinv_ref, b2_ref, out_ref):
    z = (dinv_ref[...] * (q_ref[0, 0:N, 0:C_OUT] + q_ref[1, 0:N, 0:C_OUT]
                          + ht2_ref[0:N, 0:C_OUT])
         + b2_ref[...])
    m = jnp.max(z, axis=1, keepdims=True)
    zz = z - m
    out_ref[...] = zz - jnp.log(jnp.sum(jnp.exp(zz), axis=1, keepdims=True))


def _pad_edges(a, pad):
    return jnp.concatenate([a, pad]).reshape(EROWS, LANE)


def kernel(x, edge_index, edge_attr, W1, b1, W2, b2):
    pad_idx = jnp.arange(EPAD - E, dtype=jnp.int32) % N
    src2 = _pad_edges(edge_index[0], pad_idx)
    dst2 = _pad_edges(edge_index[1], pad_idx)
    ew2 = _pad_edges(edge_attr, jnp.zeros((EPAD - E,), jnp.float32))

    deg0, deg1 = _deg_kernel(dst2, ew2)
    deg0 = deg0.reshape(NPAD, 1)
    deg1 = deg1.reshape(NPAD, 1)

    ht1, dinv = pl.pallas_call(
        _tc1_body,
        out_shape=(jax.ShapeDtypeStruct((N, H_MID), jnp.float32),
                   jax.ShapeDtypeStruct((N, 1), jnp.float32)),
    )(deg0, deg1, x, W1)

    p1 = _agg64(ht1, src2, dst2, ew2)

    ht2 = pl.pallas_call(
        _tc2_body,
        out_shape=jax.ShapeDtypeStruct((N, FW2), jnp.float32),
    )(p1, ht1, dinv, b1.reshape(1, H_MID), W2)

    p2 = _agg40(ht2, src2, dst2, ew2)

    return pl.pallas_call(
        _tc3_body,
        out_shape=jax.ShapeDtypeStruct((N, C_OUT), jnp.float32),
    )(p2, ht2, dinv, b2.reshape(1, C_OUT))


# speedup vs baseline: 1.7504x; 1.3914x over previous
import functools

import jax
import jax.numpy as jnp
from jax import lax
from jax.experimental import pallas as pl
from jax.experimental.pallas import tpu as pltpu
from jax.experimental.pallas import tpu_sc as plsc

N = 10000
E = 320000
D_IN = 128
H_MID = 64
C_OUT = 40

LANE = 128
EPAD = 327680
EROWS = EPAD // LANE
NPAD = 10240

NC = 2
NS = 16
TROWS_AGG = EROWS // (NC * NS)
TROWS_DEG = EROWS // NS
NROWS_TILE = NPAD // NS
ZROWS = 32

_MESH = plsc.VectorSubcoreMesh(core_axis_name="c", subcore_axis_name="s")
_SC_PARAMS = pltpu.CompilerParams(use_tc_tiling_on_sc=False)


@functools.partial(
    pl.kernel,
    out_type=(jax.ShapeDtypeStruct((NPAD,), jnp.float32),
              jax.ShapeDtypeStruct((NPAD,), jnp.float32)),
    mesh=_MESH,
    compiler_params=_SC_PARAMS,
    scratch_types=[
        pltpu.VMEM((TROWS_AGG, LANE), jnp.int32),
        pltpu.VMEM((TROWS_AGG, LANE), jnp.float32),
        pltpu.VMEM((NROWS_TILE,), jnp.float32),
        pltpu.VMEM_SHARED((NPAD,), jnp.float32),
        pltpu.SemaphoreType.DMA,
    ],
)
def _deg_kernel(dst_hbm, ew_hbm, deg0_out, deg1_out, idx_v, ew_v, zbuf, acc,
                sem):
    c = lax.axis_index("c")
    s = lax.axis_index("s")
    wid = c * NS + s

    zeros16 = jnp.zeros((16,), jnp.float32)

    def _zero(i, carry):
        zbuf[pl.ds(i * 16, 16)] = zeros16
        return carry

    lax.fori_loop(0, NROWS_TILE // 16, _zero, 0)
    pltpu.sync_copy(zbuf, acc.at[pl.ds(s * NROWS_TILE, NROWS_TILE)])
    row0 = wid * TROWS_AGG
    pltpu.sync_copy(dst_hbm.at[pl.ds(row0, TROWS_AGG)], idx_v)
    pltpu.sync_copy(ew_hbm.at[pl.ds(row0, TROWS_AGG)], ew_v)
    plsc.subcore_barrier()

    def _chunk(k, carry):
        cps = [
            pltpu.async_copy(
                ew_v.at[k * 8 + j], acc.at[idx_v.at[k * 8 + j]], sem,
                add=True)
            for j in range(8)
        ]
        for cp in cps:
            cp.wait()
        return carry

    lax.fori_loop(0, TROWS_AGG // 8, _chunk, 0)
    plsc.subcore_barrier()

    sl = pl.ds(s * NROWS_TILE, NROWS_TILE)

    @pl.when(c == 0)
    def _writeout0():
        pltpu.sync_copy(acc.at[sl], deg0_out.at[sl])

    @pl.when(c == 1)
    def _writeout1():
        pltpu.sync_copy(acc.at[sl], deg1_out.at[sl])


FPAD = 128
FW2 = (C_OUT + 15) // 16 * 16


def _make_agg_kernel_v2(F):
    FW = (F + 15) // 16 * 16
    nfv = FW // 16
    nk = TROWS_AGG

    @functools.partial(
        pl.kernel,
        out_type=jax.ShapeDtypeStruct((NC, NPAD, FPAD), jnp.float32),
        mesh=_MESH,
        compiler_params=_SC_PARAMS,
        scratch_types=[
            pltpu.VMEM((TROWS_AGG, LANE), jnp.int32),
            pltpu.VMEM((TROWS_AGG, LANE), jnp.int32),
            pltpu.VMEM((TROWS_AGG, LANE), jnp.float32),
            pltpu.VMEM((4, LANE, FW), jnp.float32),
            pltpu.VMEM((ZROWS, FW), jnp.float32),
            pltpu.VMEM_SHARED((NPAD, FW), jnp.float32),
            pltpu.SemaphoreType.DMA,
            pltpu.SemaphoreType.DMA,
            pltpu.SemaphoreType.DMA,
            pltpu.SemaphoreType.DMA,
            pltpu.SemaphoreType.DMA,
            pltpu.SemaphoreType.DMA,
            pltpu.SemaphoreType.DMA,
            pltpu.SemaphoreType.DMA,
        ],
    )
    def _agg(ht_hbm, src_hbm, dst_hbm, ew_hbm, out_hbm,
             src_v, dst_v, ew_v, rows, zbuf, acc,
             g0, g1, g2, g3, s0, s1, s2, s3):
        c = lax.axis_index("c")
        s = lax.axis_index("s")
        wid = c * NS + s
        gsem = (g0, g1, g2, g3)
        ssem = (s0, s1, s2, s3)
        zeros16 = jnp.zeros((16,), jnp.float32)

        def _fire_gather(k, b):
            pltpu.async_copy(
                ht_hbm.at[src_v.at[k]], rows.at[b], gsem[b])

        def _wait_gather(k, b):
            pltpu.make_async_copy(
                ht_hbm.at[src_v.at[k]], rows.at[b], gsem[b]).wait()

        def _fire_scatter(k, b):
            pltpu.async_copy(
                rows.at[b], acc.at[dst_v.at[k]], ssem[b], add=True)

        def _wait_scatter(k, b):
            pltpu.make_async_copy(
                rows.at[b], acc.at[dst_v.at[k]], ssem[b]).wait()

        def _scale(k, b):
            def body(g, carry):
                ew16 = ew_v[k, pl.ds(g * 16, 16)]
                for ii in range(16):
                    bc = lax.broadcast(ew16[ii], (16,))
                    r = g * 16 + ii
                    for f in range(nfv):
                        sl = pl.ds(f * 16, 16)
                        rows[b, r, sl] = rows[b, r, sl] * bc
                return carry

            lax.fori_loop(0, LANE // 16, body, 0, unroll=4)

        def _zero(i, carry):
            r = i // nfv
            f = i % nfv
            zbuf[r, pl.ds(f * 16, 16)] = zeros16
            return carry

        lax.fori_loop(0, ZROWS * nfv, _zero, 0)
        zcps = [
            pltpu.make_async_copy(
                zbuf, acc.at[pl.ds(s * NROWS_TILE + q * ZROWS, ZROWS)], g0)
            for q in range(NROWS_TILE // ZROWS)
        ]
        for cp in zcps:
            cp.start()
        row0 = wid * TROWS_AGG
        pltpu.sync_copy(src_hbm.at[pl.ds(row0, TROWS_AGG)], src_v)
        pltpu.sync_copy(dst_hbm.at[pl.ds(row0, TROWS_AGG)], dst_v)
        pltpu.sync_copy(ew_hbm.at[pl.ds(row0, TROWS_AGG)], ew_v)
        for cp in zcps:
            cp.wait()
        _fire_gather(0, 0)
        _fire_gather(1, 1)
        plsc.subcore_barrier()

        _fire_gather(2, 2)
        _wait_gather(0, 0)
        _scale(0, 0)
        _fire_scatter(0, 0)

        _fire_gather(3, 3)
        _wait_gather(1, 1)
        _scale(1, 1)
        _fire_scatter(1, 1)

        def _steady(k0, carry):
            for boff in range(4):
                k = 4 * k0 + 2 + boff
                b = (2 + boff) % 4
                bn = boff
                _wait_scatter(k - 2, bn)
                _fire_gather(k + 2, bn)
                _wait_gather(k, b)
                _scale(k, b)
                _fire_scatter(k, b)
            return carry

        lax.fori_loop(0, (nk - 4) // 4, _steady, 0)

        _wait_scatter(nk - 4, 0)
        _wait_gather(nk - 2, 2)
        _scale(nk - 2, 2)
        _fire_scatter(nk - 2, 2)

        _wait_scatter(nk - 3, 1)
        _wait_gather(nk - 1, 3)
        _scale(nk - 1, 3)
        _fire_scatter(nk - 1, 3)

        _wait_scatter(nk - 2, 2)
        _wait_scatter(nk - 1, 3)
        plsc.subcore_barrier()

        sl = pl.ds(s * NROWS_TILE, NROWS_TILE)
        pltpu.sync_copy(acc.at[sl], out_hbm.at[c, sl, pl.ds(0, FW)])

    return _agg


def _make_agg_kernel(F):
    cb_rows = 2
    cb = cb_rows * LANE

    @functools.partial(
        pl.kernel,
        out_type=jax.ShapeDtypeStruct((NC, NPAD, FPAD), jnp.float32),
        mesh=_MESH,
        compiler_params=_SC_PARAMS,
        scratch_types=[
            pltpu.VMEM((cb_rows, LANE), jnp.int32),
            pltpu.VMEM((cb_rows, LANE), jnp.int32),
            pltpu.VMEM((cb_rows, LANE), jnp.float32),
            pltpu.VMEM((cb, FPAD), jnp.float32),
            pltpu.VMEM((ZROWS, FPAD), jnp.float32),
            pltpu.VMEM_SHARED((NPAD, FPAD), jnp.float32),
            pltpu.SemaphoreType.DMA,
        ],
    )
    def _agg(ht_hbm, src_hbm, dst_hbm, ew_hbm, out_hbm,
             src_v, dst_v, ew_v, rows, zbuf, acc, sem):
        c = lax.axis_index("c")
        s = lax.axis_index("s")
        wid = c * NS + s

        zeros16 = jnp.zeros((16,), jnp.float32)

        def _zero(i, carry):
            r = i // (FPAD // 16)
            f = i % (FPAD // 16)
            zbuf[r, pl.ds(f * 16, 16)] = zeros16
            return carry

        lax.fori_loop(0, ZROWS * (FPAD // 16), _zero, 0)
        for q in range(NROWS_TILE // ZROWS):
            pltpu.sync_copy(
                zbuf, acc.at[pl.ds(s * NROWS_TILE + q * ZROWS, ZROWS)])
        plsc.subcore_barrier()

        row0 = wid * TROWS_AGG

        def _chunk(k, carry):
            base = row0 + k * cb_rows
            pltpu.sync_copy(src_hbm.at[pl.ds(base, cb_rows)], src_v)
            pltpu.sync_copy(dst_hbm.at[pl.ds(base, cb_rows)], dst_v)
            pltpu.sync_copy(ew_hbm.at[pl.ds(base, cb_rows)], ew_v)
            cps = [
                pltpu.async_copy(
                    ht_hbm.at[src_v.at[j]],
                    rows.at[pl.ds(j * LANE, LANE)], sem)
                for j in range(cb_rows)
            ]
            for cp in cps:
                cp.wait()

            def _scale(i, carry2):
                j = i % cb_rows
                g = i // cb_rows
                ew16 = ew_v[j, pl.ds(g * 16, 16)]
                for ii in range(16):
                    bc = lax.broadcast(ew16[ii], (16,))
                    r = j * LANE + g * 16 + ii
                    for f in range((F + 15) // 16):
                        sl = pl.ds(f * 16, 16)
                        rows[r, sl] = rows[r, sl] * bc
                return carry2

            lax.fori_loop(0, cb_rows * (LANE // 16), _scale, 0)

            cps = [
                pltpu.async_copy(
                    rows.at[pl.ds(j * LANE, LANE)],
                    acc.at[dst_v.at[j]], sem, add=True)
                for j in range(cb_rows)
            ]
            for cp in cps:
                cp.wait()
            return carry

        lax.fori_loop(0, TROWS_AGG // cb_rows, _chunk, 0)
        plsc.subcore_barrier()

        sl = pl.ds(s * NROWS_TILE, NROWS_TILE)
        pltpu.sync_copy(acc.at[sl], out_hbm.at[c, sl])

    return _agg


_agg64 = _make_agg_kernel_v2(H_MID)
_agg40 = _make_agg_kernel_v2(C_OUT)


def _tc1_body(deg0_ref, deg1_ref, x_ref, w1_ref, ht_ref, dinv_ref):
    deg = deg0_ref[0:N, :] + deg1_ref[0:N, :] + 1.0
    dinv = jnp.where(deg > 0, lax.rsqrt(jnp.maximum(deg, 1e-12)), 0.0)
    h = jnp.dot(x_ref[...], w1_ref[...], preferred_element_type=jnp.float32)
    ht_ref[...] = dinv * h
    dinv_ref[...] = dinv


def _tc2_body(p_ref, ht1_ref, dinv_ref, b1_ref, w2_ref, ht2_ref):
    dinv = dinv_ref[...]
    agg = (p_ref[0, 0:N, 0:H_MID] + p_ref[1, 0:N, 0:H_MID]
           + ht1_ref[0:N, 0:H_MID])
    x2 = jnp.maximum(dinv * agg + b1_ref[...], 0.0)
    ht2 = dinv * jnp.dot(x2, w2_ref[...], preferred_element_type=jnp.float32)
    ht2_ref[...] = jnp.concatenate(
        [ht2, jnp.zeros((N, FW2 - C_OUT), jnp.float32)], axis=1)


def _tc3_body(q_ref, ht2_ref, dinv_ref, b2_ref, out_ref):
    z = (dinv_ref[...] * (q_ref[0, 0:N, 0:C_OUT] + q_ref[1, 0:N, 0:C_OUT]
                          + ht2_ref[0:N, 0:C_OUT])
         + b2_ref[...])
    m = jnp.max(z, axis=1, keepdims=True)
    zz = z - m
    out_ref[...] = zz - jnp.log(jnp.sum(jnp.exp(zz), axis=1, keepdims=True))


def _pad_edges(a, pad):
    return jnp.concatenate([a, pad]).reshape(EROWS, LANE)


def kernel(x, edge_index, edge_attr, W1, b1, W2, b2):
    pad_idx = jnp.arange(EPAD - E, dtype=jnp.int32) % N
    src2 = _pad_edges(edge_index[0], pad_idx)
    dst2 = _pad_edges(edge_index[1], pad_idx)
    ew2 = _pad_edges(edge_attr, jnp.zeros((EPAD - E,), jnp.float32))

    deg0, deg1 = _deg_kernel(dst2, ew2)
    deg0 = deg0.reshape(NPAD, 1)
    deg1 = deg1.reshape(NPAD, 1)

    ht1, dinv = pl.pallas_call(
        _tc1_body,
        out_shape=(jax.ShapeDtypeStruct((N, H_MID), jnp.float32),
                   jax.ShapeDtypeStruct((N, 1), jnp.float32)),
    )(deg0, deg1, x, W1)

    p1 = _agg64(ht1, src2, dst2, ew2)

    ht2 = pl.pallas_call(
        _tc2_body,
        out_shape=jax.ShapeDtypeStruct((N, FW2), jnp.float32),
    )(p1, ht1, dinv, b1.reshape(1, H_MID), W2)

    p2 = _agg40(ht2, src2, dst2, ew2)

    return pl.pallas_call(
        _tc3_body,
        out_shape=jax.ShapeDtypeStruct((N, C_OUT), jnp.float32),
    )(p2, ht2, dinv, b2.reshape(1, C_OUT))
